# baseline (speedup 1.0000x reference)
import functools

import jax
import jax.numpy as jnp
from jax import lax
from jax.experimental import pallas as pl
from jax.experimental.pallas import tpu as pltpu
from jax.experimental.pallas import tpu_sc as plsc

TM = 512
IB = 768


def _sc_gather(table, idx):
    rows, dim = table.shape
    (m,) = idx.shape
    info = plsc.get_sparse_core_info()
    nw = info.num_cores * info.num_subcores
    m_per_w = m // nw
    chunk = 128
    n_chunks = m_per_w // chunk
    mesh = plsc.VectorSubcoreMesh(core_axis_name="c", subcore_axis_name="s")

    @functools.partial(
        pl.kernel,
        mesh=mesh,
        out_type=jax.ShapeDtypeStruct((m, dim), table.dtype),
        scratch_types=[
            pltpu.VMEM((chunk,), jnp.int32),
            pltpu.VMEM((chunk, dim), table.dtype),
            pltpu.SemaphoreType.DMA,
        ],
    )
    def k(table_hbm, idx_hbm, out_hbm, idx_v, rows_v, sem):
        wid = lax.axis_index("s") * info.num_cores + lax.axis_index("c")
        base = wid * m_per_w
        for c in range(n_chunks):
            off = base + c * chunk
            pltpu.sync_copy(idx_hbm.at[pl.ds(off, chunk)], idx_v)
            pltpu.async_copy(table_hbm.at[idx_v], rows_v, sem).wait()
            pltpu.sync_copy(rows_v, out_hbm.at[pl.ds(off, chunk)])

    return k(table, idx)


def _ffn_body(tile_a, eidx_a, first_a, start_a, end_a,
              xs_ref, w1_ref, b1_ref, w2_ref, b2_ref, out_ref):
    w = pl.program_id(0)
    k = pl.program_id(1)
    tile = tile_a[w]
    row = tile * TM + lax.broadcasted_iota(jnp.int32, (TM, 1), 0)
    msk = (row >= start_a[w]) & (row < end_a[w])

    x = xs_ref[...].astype(jnp.bfloat16)
    h = lax.dot_general(x, w1_ref[0].astype(jnp.bfloat16),
                        (((1,), (1,)), ((), ())),
                        preferred_element_type=jnp.float32)
    h = h + b1_ref[0]
    g = 0.5 * h * (1.0 + lax.erf(h * 0.7071067811865476))
    p = lax.dot_general(g.astype(jnp.bfloat16),
                        w2_ref[0].astype(jnp.bfloat16),
                        (((1,), (1,)), ((), ())),
                        preferred_element_type=jnp.float32)
    p = p + jnp.where(k == 0, b2_ref[0], 0.0)
    contrib = jnp.where(msk, p, 0.0)

    first = (first_a[w] == 1) & (k == 0)

    @pl.when(first)
    def _():
        out_ref[...] = contrib

    @pl.when(jnp.logical_not(first))
    def _():
        out_ref[...] += contrib


def _grouped_ffn(x_sorted, w1, b1, w2, b2, tile_a, eidx_a, first_a,
                 start_a, end_a, n_units):
    n, d = x_sorted.shape
    e, inner, _ = w1.shape
    kk = inner // IB
    b1 = b1.reshape(e * kk, 1, IB)
    b2 = b2.reshape(e, 1, d)
    grid_spec = pltpu.PrefetchScalarGridSpec(
        num_scalar_prefetch=5,
        grid=(n_units, kk),
        in_specs=[
            pl.BlockSpec((TM, d), lambda w, k, t, ei, f, s, en: (t[w], 0)),
            pl.BlockSpec((1, IB, d), lambda w, k, t, ei, f, s, en: (ei[w], k, 0)),
            pl.BlockSpec((1, 1, IB),
                         lambda w, k, t, ei, f, s, en: (ei[w] * kk + k, 0, 0)),
            pl.BlockSpec((1, d, IB), lambda w, k, t, ei, f, s, en: (ei[w], 0, k)),
            pl.BlockSpec((1, 1, d), lambda w, k, t, ei, f, s, en: (ei[w], 0, 0)),
        ],
        out_specs=pl.BlockSpec((TM, d), lambda w, k, t, ei, f, s, en: (t[w], 0)),
    )
    return pl.pallas_call(
        _ffn_body,
        grid_spec=grid_spec,
        out_shape=jax.ShapeDtypeStruct((n, d), jnp.float32),
        compiler_params=pltpu.CompilerParams(
            dimension_semantics=("arbitrary", "arbitrary")),
    )(tile_a, eidx_a, first_a, start_a, end_a, x_sorted, w1, b1, w2, b2)


def kernel(x, Wg, W1, b1, W2, b2):
    b, n, d = x.shape
    e = Wg.shape[0]
    x_flat = x.reshape(-1, d)
    nt = x_flat.shape[0]
    n_tiles = nt // TM
    n_units = n_tiles + e - 1

    gating_logits = x_flat @ Wg.T
    gating_probs = jax.nn.softmax(gating_logits, axis=-1)
    expert_usage = gating_probs.mean(0)
    expert_prob_dist = gating_probs.sum(0)
    load_balancing_loss = e * jnp.sum(expert_usage * expert_prob_dist)
    _, top1 = lax.top_k(gating_logits, 1)
    ids = top1[:, 0].astype(jnp.int32)

    perm = jnp.argsort(ids).astype(jnp.int32)
    inv_perm = jnp.zeros((nt,), jnp.int32).at[perm].set(
        jnp.arange(nt, dtype=jnp.int32))
    counts = jnp.bincount(ids, length=e)
    ends = jnp.cumsum(counts)
    starts = ends - counts
    t_lo = starts // TM
    t_hi = jnp.where(counts > 0, (ends - 1) // TM, t_lo)
    ntiles = jnp.where(counts > 0, t_hi - t_lo + 1, 0)
    unit_end = jnp.cumsum(ntiles)
    unit_start = unit_end - ntiles
    total = unit_end[-1]
    wix = jnp.arange(n_units)
    e_of = jnp.minimum(
        jnp.searchsorted(unit_end, wix, side="right"), e - 1).astype(jnp.int32)
    tile_of = (t_lo[e_of] + (wix - unit_start[e_of])).astype(jnp.int32)
    valid = wix < total
    last = total - 1
    e_last = jnp.minimum(
        jnp.searchsorted(unit_end, last, side="right"), e - 1).astype(jnp.int32)
    tile_last = (t_lo[e_last] + (last - unit_start[e_last])).astype(jnp.int32)
    e_of = jnp.where(valid, e_of, e_last)
    tile_of = jnp.where(valid, tile_of, tile_last)
    start_of = jnp.where(valid, starts[e_of], 0).astype(jnp.int32)
    end_of = jnp.where(valid, ends[e_of], 0).astype(jnp.int32)
    first_of = jnp.concatenate(
        [jnp.ones((1,), jnp.int32),
         (tile_of[1:] != tile_of[:-1]).astype(jnp.int32)])

    x_sorted = _sc_gather(x_flat, perm)
    y_sorted = _grouped_ffn(x_sorted, W1, b1, W2, b2, tile_of, e_of,
                            first_of, start_of, end_of, n_units)
    out = _sc_gather(y_sorted, inv_perm)
    return out.reshape(b, n, d), load_balancing_loss


# speedup vs baseline: 1.0280x; 1.0280x over previous
import functools

import jax
import jax.numpy as jnp
from jax import lax
from jax.experimental import pallas as pl
from jax.experimental.pallas import tpu as pltpu
from jax.experimental.pallas import tpu_sc as plsc

TM = 512
IB = 768


def _sc_gather(table, idx):
    rows, dim = table.shape
    (m,) = idx.shape
    info = plsc.get_sparse_core_info()
    nw = info.num_cores * info.num_subcores
    m_per_w = m // nw
    chunk = 128
    n_chunks = m_per_w // chunk
    mesh = plsc.VectorSubcoreMesh(core_axis_name="c", subcore_axis_name="s")

    @functools.partial(
        pl.kernel,
        mesh=mesh,
        out_type=jax.ShapeDtypeStruct((m, dim), table.dtype),
        scratch_types=[
            pltpu.VMEM((chunk,), jnp.int32),
            pltpu.VMEM((chunk, dim), table.dtype),
            pltpu.SemaphoreType.DMA,
        ],
    )
    def k(table_hbm, idx_hbm, out_hbm, idx_v, rows_v, sem):
        wid = lax.axis_index("s") * info.num_cores + lax.axis_index("c")
        base = wid * m_per_w
        for c in range(n_chunks):
            off = base + c * chunk
            pltpu.sync_copy(idx_hbm.at[pl.ds(off, chunk)], idx_v)
            pltpu.async_copy(table_hbm.at[idx_v], rows_v, sem).wait()
            pltpu.sync_copy(rows_v, out_hbm.at[pl.ds(off, chunk)])

    return k(table, idx)


def _ffn_body(tile_a, eidx_a, first_a, start_a, end_a,
              xs_ref, w1_ref, b1_ref, w2_ref, b2_ref, out_ref):
    w = pl.program_id(0)
    k = pl.program_id(1)
    tile = tile_a[w]
    row = tile * TM + lax.broadcasted_iota(jnp.int32, (TM, 1), 0)
    msk = (row >= start_a[w]) & (row < end_a[w])

    x = xs_ref[...].astype(jnp.bfloat16)
    h = lax.dot_general(x, w1_ref[0].astype(jnp.bfloat16),
                        (((1,), (1,)), ((), ())),
                        preferred_element_type=jnp.float32)
    h = h + b1_ref[0]
    g = 0.5 * h * (1.0 + lax.erf(h * 0.7071067811865476))
    p = lax.dot_general(g.astype(jnp.bfloat16),
                        w2_ref[0].astype(jnp.bfloat16),
                        (((1,), (1,)), ((), ())),
                        preferred_element_type=jnp.float32)
    p = p + jnp.where(k == 0, b2_ref[0], 0.0)
    contrib = jnp.where(msk, p, 0.0)

    first = (first_a[w] == 1) & (k == 0)

    @pl.when(first)
    def _():
        out_ref[...] = contrib

    @pl.when(jnp.logical_not(first))
    def _():
        out_ref[...] += contrib


def _grouped_ffn(x_sorted, w1, b1, w2, b2, tile_a, eidx_a, first_a,
                 start_a, end_a, n_units):
    n, d = x_sorted.shape
    e, inner, _ = w1.shape
    kk = inner // IB
    b1 = b1.reshape(e * kk, 1, IB)
    b2 = b2.reshape(e, 1, d)
    grid_spec = pltpu.PrefetchScalarGridSpec(
        num_scalar_prefetch=5,
        grid=(n_units, kk),
        in_specs=[
            pl.BlockSpec((TM, d), lambda w, k, t, ei, f, s, en: (t[w], 0)),
            pl.BlockSpec((1, IB, d), lambda w, k, t, ei, f, s, en: (ei[w], k, 0)),
            pl.BlockSpec((1, 1, IB),
                         lambda w, k, t, ei, f, s, en: (ei[w] * kk + k, 0, 0)),
            pl.BlockSpec((1, d, IB), lambda w, k, t, ei, f, s, en: (ei[w], 0, k)),
            pl.BlockSpec((1, 1, d), lambda w, k, t, ei, f, s, en: (ei[w], 0, 0)),
        ],
        out_specs=pl.BlockSpec((TM, d), lambda w, k, t, ei, f, s, en: (t[w], 0)),
    )
    return pl.pallas_call(
        _ffn_body,
        grid_spec=grid_spec,
        out_shape=jax.ShapeDtypeStruct((n, d), jnp.float32),
        compiler_params=pltpu.CompilerParams(
            dimension_semantics=("arbitrary", "arbitrary")),
    )(tile_a, eidx_a, first_a, start_a, end_a, x_sorted, w1, b1, w2, b2)


def kernel(x, Wg, W1, b1, W2, b2):
    b, n, d = x.shape
    e = Wg.shape[0]
    x_flat = x.reshape(-1, d)
    nt = x_flat.shape[0]
    n_tiles = nt // TM
    n_units = n_tiles + e - 1

    gating_logits = x_flat @ Wg.T
    gating_probs = jax.nn.softmax(gating_logits, axis=-1)
    expert_usage = gating_probs.mean(0)
    expert_prob_dist = gating_probs.sum(0)
    load_balancing_loss = e * jnp.sum(expert_usage * expert_prob_dist)
    ids = jnp.argmax(gating_logits, axis=-1).astype(jnp.int32)

    onehot = (ids[:, None] == jnp.arange(e, dtype=jnp.int32)[None, :])
    cum = jnp.cumsum(onehot.astype(jnp.int32), axis=0)
    counts = cum[-1]
    rank = jnp.take_along_axis(cum, ids[:, None], axis=1)[:, 0] - 1
    starts0 = jnp.cumsum(counts) - counts
    position = (starts0[ids] + rank).astype(jnp.int32)
    perm = jnp.zeros((nt,), jnp.int32).at[position].set(
        jnp.arange(nt, dtype=jnp.int32))
    inv_perm = position
    ends = jnp.cumsum(counts)
    starts = ends - counts
    t_lo = starts // TM
    t_hi = jnp.where(counts > 0, (ends - 1) // TM, t_lo)
    ntiles = jnp.where(counts > 0, t_hi - t_lo + 1, 0)
    unit_end = jnp.cumsum(ntiles)
    unit_start = unit_end - ntiles
    total = unit_end[-1]
    wix = jnp.arange(n_units)
    e_of = jnp.minimum(
        jnp.searchsorted(unit_end, wix, side="right"), e - 1).astype(jnp.int32)
    tile_of = (t_lo[e_of] + (wix - unit_start[e_of])).astype(jnp.int32)
    valid = wix < total
    last = total - 1
    e_last = jnp.minimum(
        jnp.searchsorted(unit_end, last, side="right"), e - 1).astype(jnp.int32)
    tile_last = (t_lo[e_last] + (last - unit_start[e_last])).astype(jnp.int32)
    e_of = jnp.where(valid, e_of, e_last)
    tile_of = jnp.where(valid, tile_of, tile_last)
    start_of = jnp.where(valid, starts[e_of], 0).astype(jnp.int32)
    end_of = jnp.where(valid, ends[e_of], 0).astype(jnp.int32)
    first_of = jnp.concatenate(
        [jnp.ones((1,), jnp.int32),
         (tile_of[1:] != tile_of[:-1]).astype(jnp.int32)])

    x_sorted = _sc_gather(x_flat, perm)
    y_sorted = _grouped_ffn(x_sorted, W1, b1, W2, b2, tile_of, e_of,
                            first_of, start_of, end_of, n_units)
    out = _sc_gather(y_sorted, inv_perm)
    return out.reshape(b, n, d), load_balancing_loss


# speedup vs baseline: 1.0784x; 1.0491x over previous
import functools

import jax
import jax.numpy as jnp
from jax import lax
from jax.experimental import pallas as pl
from jax.experimental.pallas import tpu as pltpu
from jax.experimental.pallas import tpu_sc as plsc

TM = 1024
IB = 1536


def _sc_gather(table, idx):
    rows, dim = table.shape
    (m,) = idx.shape
    info = plsc.get_sparse_core_info()
    nw = info.num_cores * info.num_subcores
    m_per_w = m // nw
    chunk = 128
    n_chunks = m_per_w // chunk
    mesh = plsc.VectorSubcoreMesh(core_axis_name="c", subcore_axis_name="s")

    @functools.partial(
        pl.kernel,
        mesh=mesh,
        out_type=jax.ShapeDtypeStruct((m, dim), table.dtype),
        scratch_types=[
            pltpu.VMEM((chunk,), jnp.int32),
            pltpu.VMEM((chunk, dim), table.dtype),
            pltpu.SemaphoreType.DMA,
        ],
    )
    def k(table_hbm, idx_hbm, out_hbm, idx_v, rows_v, sem):
        wid = lax.axis_index("s") * info.num_cores + lax.axis_index("c")
        base = wid * m_per_w
        for c in range(n_chunks):
            off = base + c * chunk
            pltpu.sync_copy(idx_hbm.at[pl.ds(off, chunk)], idx_v)
            pltpu.async_copy(table_hbm.at[idx_v], rows_v, sem).wait()
            pltpu.sync_copy(rows_v, out_hbm.at[pl.ds(off, chunk)])

    return k(table, idx)


def _ffn_body(tile_a, eidx_a, first_a, start_a, end_a,
              xs_ref, w1_ref, b1_ref, w2_ref, b2_ref, out_ref):
    w = pl.program_id(0)
    k = pl.program_id(1)
    tile = tile_a[w]
    row = tile * TM + lax.broadcasted_iota(jnp.int32, (TM, 1), 0)
    msk = (row >= start_a[w]) & (row < end_a[w])

    x = xs_ref[...].astype(jnp.bfloat16)
    h = lax.dot_general(x, w1_ref[0].astype(jnp.bfloat16),
                        (((1,), (1,)), ((), ())),
                        preferred_element_type=jnp.float32)
    h = h + b1_ref[0]
    g = 0.5 * h * (1.0 + lax.erf(h * 0.7071067811865476))
    p = lax.dot_general(g.astype(jnp.bfloat16),
                        w2_ref[0].astype(jnp.bfloat16),
                        (((1,), (1,)), ((), ())),
                        preferred_element_type=jnp.float32)
    p = p + jnp.where(k == 0, b2_ref[0], 0.0)
    contrib = jnp.where(msk, p, 0.0)

    first = (first_a[w] == 1) & (k == 0)

    @pl.when(first)
    def _():
        out_ref[...] = contrib

    @pl.when(jnp.logical_not(first))
    def _():
        out_ref[...] += contrib


def _grouped_ffn(x_sorted, w1, b1, w2, b2, tile_a, eidx_a, first_a,
                 start_a, end_a, n_units):
    n, d = x_sorted.shape
    e, inner, _ = w1.shape
    kk = inner // IB
    b1 = b1.reshape(e * kk, 1, IB)
    b2 = b2.reshape(e, 1, d)
    grid_spec = pltpu.PrefetchScalarGridSpec(
        num_scalar_prefetch=5,
        grid=(n_units, kk),
        in_specs=[
            pl.BlockSpec((TM, d), lambda w, k, t, ei, f, s, en: (t[w], 0)),
            pl.BlockSpec((1, IB, d), lambda w, k, t, ei, f, s, en: (ei[w], k, 0)),
            pl.BlockSpec((1, 1, IB),
                         lambda w, k, t, ei, f, s, en: (ei[w] * kk + k, 0, 0)),
            pl.BlockSpec((1, d, IB), lambda w, k, t, ei, f, s, en: (ei[w], 0, k)),
            pl.BlockSpec((1, 1, d), lambda w, k, t, ei, f, s, en: (ei[w], 0, 0)),
        ],
        out_specs=pl.BlockSpec((TM, d), lambda w, k, t, ei, f, s, en: (t[w], 0)),
    )
    return pl.pallas_call(
        _ffn_body,
        grid_spec=grid_spec,
        out_shape=jax.ShapeDtypeStruct((n, d), jnp.float32),
        compiler_params=pltpu.CompilerParams(
            dimension_semantics=("arbitrary", "arbitrary")),
    )(tile_a, eidx_a, first_a, start_a, end_a, x_sorted, w1, b1, w2, b2)


def kernel(x, Wg, W1, b1, W2, b2):
    b, n, d = x.shape
    e = Wg.shape[0]
    x_flat = x.reshape(-1, d)
    nt = x_flat.shape[0]
    n_tiles = nt // TM
    n_units = n_tiles + e - 1

    gating_logits = x_flat @ Wg.T
    gating_probs = jax.nn.softmax(gating_logits, axis=-1)
    expert_usage = gating_probs.mean(0)
    expert_prob_dist = gating_probs.sum(0)
    load_balancing_loss = e * jnp.sum(expert_usage * expert_prob_dist)
    ids = jnp.argmax(gating_logits, axis=-1).astype(jnp.int32)

    onehot = (ids[:, None] == jnp.arange(e, dtype=jnp.int32)[None, :])
    cum = jnp.cumsum(onehot.astype(jnp.int32), axis=0)
    counts = cum[-1]
    rank = jnp.take_along_axis(cum, ids[:, None], axis=1)[:, 0] - 1
    starts0 = jnp.cumsum(counts) - counts
    position = (starts0[ids] + rank).astype(jnp.int32)
    perm = jnp.zeros((nt,), jnp.int32).at[position].set(
        jnp.arange(nt, dtype=jnp.int32))
    inv_perm = position
    ends = jnp.cumsum(counts)
    starts = ends - counts
    t_lo = starts // TM
    t_hi = jnp.where(counts > 0, (ends - 1) // TM, t_lo)
    ntiles = jnp.where(counts > 0, t_hi - t_lo + 1, 0)
    unit_end = jnp.cumsum(ntiles)
    unit_start = unit_end - ntiles
    total = unit_end[-1]
    wix = jnp.arange(n_units)
    e_of = jnp.minimum(
        jnp.searchsorted(unit_end, wix, side="right"), e - 1).astype(jnp.int32)
    tile_of = (t_lo[e_of] + (wix - unit_start[e_of])).astype(jnp.int32)
    valid = wix < total
    last = total - 1
    e_last = jnp.minimum(
        jnp.searchsorted(unit_end, last, side="right"), e - 1).astype(jnp.int32)
    tile_last = (t_lo[e_last] + (last - unit_start[e_last])).astype(jnp.int32)
    e_of = jnp.where(valid, e_of, e_last)
    tile_of = jnp.where(valid, tile_of, tile_last)
    start_of = jnp.where(valid, starts[e_of], 0).astype(jnp.int32)
    end_of = jnp.where(valid, ends[e_of], 0).astype(jnp.int32)
    first_of = jnp.concatenate(
        [jnp.ones((1,), jnp.int32),
         (tile_of[1:] != tile_of[:-1]).astype(jnp.int32)])

    x_sorted = _sc_gather(x_flat, perm)
    y_sorted = _grouped_ffn(x_sorted, W1, b1, W2, b2, tile_of, e_of,
                            first_of, start_of, end_of, n_units)
    out = _sc_gather(y_sorted, inv_perm)
    return out.reshape(b, n, d), load_balancing_loss
